# baseline (speedup 1.0000x reference)
import jax
import jax.numpy as jnp
from jax import lax
from jax.experimental import pallas as pl
from jax.experimental.pallas import tpu as pltpu
from jax.experimental.pallas import tpu_sc as plsc

L = 16
NW = 32
G = 26
D = 256
CH = 400
NG = CH // L
GG = CH * G // L
M_TOTAL = 100000
NCHUNK = M_TOTAL // CH


def _sc_body(g_hbm, f_hbm, p_hbm, rt_hbm, rm_hbm, z_hbm,
             pool_out, tag_out, reu_out, unc_out, inst_out,
             pv, rtv, rmv, fm_v, um_v, zeros_v,
             g0, g1, go0, go1, r0, r1, u0, u1, t0, t1,
             *sems):
    G_V = [g0, g1]
    GO_V = [go0, go1]
    R_V = [r0, r1]
    U_V = [u0, u1]
    T_V = [t0, t1]
    SEM = [sems[0:6], sems[6:12]]

    wid = lax.axis_index("s") * 2 + lax.axis_index("c")

    pltpu.sync_copy(p_hbm, pv)
    pltpu.sync_copy(rt_hbm, rtv)
    pltpu.sync_copy(rm_hbm, rmv)
    pltpu.sync_copy(z_hbm, zeros_v)

    P = [pv[pl.ds(i * L, L)] for i in range(22)]
    (w00, w01, w02, w03, w10, w11, w12, w13, w20, w21, w22, w23,
     fx, fy, cx0, cy0, n0, n1, n2, f0b, f1b, f2b) = P

    n_my = (NCHUNK - wid + NW - 1) // NW

    def base_of(k):
        return (wid + k * NW) * CH

    def in_descs(k, b):
        base = base_of(k)
        return [
            pltpu.make_async_copy(g_hbm.at[pl.ds(base * G, CH * G)],
                                  G_V[b], SEM[b][0]),
            pltpu.make_async_copy(f_hbm.at[pl.ds(base, CH)],
                                  inst_out.at[pl.ds(base, CH)], SEM[b][1]),
        ]

    def out_descs(k, b):
        base = base_of(k)
        return [
            pltpu.make_async_copy(GO_V[b], pool_out.at[pl.ds(base * G, CH * G)],
                                  SEM[b][2]),
            pltpu.make_async_copy(R_V[b], reu_out.at[pl.ds(base * G, CH * G)],
                                  SEM[b][3]),
            pltpu.make_async_copy(U_V[b], unc_out.at[pl.ds(base * G, CH * G)],
                                  SEM[b][4]),
            pltpu.make_async_copy(T_V[b], tag_out.at[pl.ds(base, CH)],
                                  SEM[b][5]),
        ]

    def bf16_round(v):
        u = plsc.bitcast(v, jnp.int32)
        r = u + (jnp.int32(0x7FFF) + ((u >> 16) & 1))
        return plsc.bitcast(r & jnp.int32(-65536), jnp.float32)

    def compute(b):
        @plsc.parallel_loop(0, NG, unroll=2)
        def mask_body(j):
            m26 = rmv[pl.ds(j * L, L)]
            x = plsc.load_gather(G_V[b], [m26])
            y = plsc.load_gather(G_V[b], [m26 + 1])
            z = plsc.load_gather(G_V[b], [m26 + 2])
            xb = bf16_round(x)
            yb = bf16_round(y)
            zb = bf16_round(z)
            cx = xb * w00 + yb * w01 + zb * w02 + w03
            cy = xb * w10 + yb * w11 + zb * w12 + w13
            cz_ = xb * w20 + yb * w21 + zb * w22 + w23
            mask1 = cz_ > 1e-6
            cz = jnp.maximum(cz_, 1e-6)
            pxf = fx * (cx / cz) + cx0
            pyf = fy * (cy / cz) + cy0
            mask2 = ((pxf >= 0.0) & (pxf < 640.0)
                     & (pyf >= 0.0) & (pyf < 480.0))
            in_vox = ((x > n0) & (x < f0b) & (y > n1) & (y < f1b)
                      & (z > n2) & (z < f2b))
            mask_det = mask1 & mask2 & in_vox
            g23 = plsc.load_gather(G_V[b], [m26 + 23])
            g24 = plsc.load_gather(G_V[b], [m26 + 24])
            one = jnp.full((L,), 1.0, jnp.float32)
            half = jnp.full((L,), 0.5, jnp.float32)
            zero = jnp.zeros((L,), jnp.float32)
            plsc.store_scatter(G_V[b], [m26 + 24],
                               jnp.where(in_vox, one, g24))
            T_V[b][pl.ds(j * L, L)] = jnp.where(
                mask_det, jnp.where(g23 == 1.0, half, zero), one)
            fm_v[pl.ds(j * L, L)] = jnp.where(in_vox, one, zero)
            um_v[pl.ds(j * L, L)] = jnp.where(mask_det, zero, one)

        @plsc.parallel_loop(0, GG, unroll=4)
        def g_body(kk):
            off = kk * L
            row = rtv[pl.ds(off, L)]
            fmv = plsc.load_gather(fm_v, [row])
            umv = plsc.load_gather(um_v, [row])
            gv = G_V[b][pl.ds(off, L)]
            GO_V[b][pl.ds(off, L)] = gv
            R_V[b][pl.ds(off, L)] = gv * fmv
            U_V[b][pl.ds(off, L)] = gv * umv

    def zero_scatter(k):
        base = base_of(k)
        iota = lax.iota(jnp.int32, L)

        def zs_body(j, _):
            fmg = fm_v[pl.ds(j * L, L)]
            zm = fmg == 0.0
            rows0 = base + j * L

            @pl.when(jnp.any(zm))
            def _():
                ffs = plsc.all_reduce_ffs(zm)
                rows = rows0 + iota
                idx = jnp.where(zm, rows, rows0 + ffs)
                pltpu.sync_copy(zeros_v, inst_out.at[idx])

            return 0

        lax.fori_loop(0, NG, zs_body, 0)

    for d in in_descs(0, 0):
        d.start()

    def body(i, _):
        for bslot in (0, 1):
            k = i * 2 + bslot

            @pl.when(k < n_my)
            def _(k=k, bslot=bslot):
                in_descs(k, bslot)[0].wait()

                @pl.when(k >= 2)
                def _():
                    for dsc in out_descs(k - 2, bslot):
                        dsc.wait()

                @pl.when(k + 1 < n_my)
                def _():
                    for dsc in in_descs(k + 1, 1 - bslot):
                        dsc.start()

                compute(bslot)
                for dsc in out_descs(k, bslot):
                    dsc.start()
                in_descs(k, bslot)[1].wait()
                zero_scatter(k)

        return 0

    lax.fori_loop(0, (n_my + 1) // 2, body, 0)

    for bslot in (0, 1):
        for k_last in (n_my - 2, n_my - 1):
            @pl.when((k_last >= 0) & (k_last % 2 == bslot))
            def _(k_last=k_last, bslot=bslot):
                for dsc in out_descs(k_last, bslot):
                    dsc.wait()


def kernel(gaussian_pool, instance_feature_pool, world2cam, cam_k,
           vox_origin, scene_size, mlvl_img_feat, anchor):
    M = gaussian_pool.shape[0]
    batch_size = mlvl_img_feat.shape[0]
    eps = jnp.float32(0.001)
    near = vox_origin + eps
    far = vox_origin + scene_size - eps
    w_rounded = world2cam[:3, :].astype(jnp.bfloat16).astype(jnp.float32)
    params = jnp.concatenate([
        w_rounded.reshape(-1),
        jnp.stack([cam_k[0, 0], cam_k[1, 1], cam_k[0, 2], cam_k[1, 2]]),
        near, far,
    ]).astype(jnp.float32)
    params_b = jnp.repeat(params, L)

    rowtab = jnp.repeat(jnp.arange(CH, dtype=jnp.int32), G)
    rowm26 = jnp.arange(CH, dtype=jnp.int32) * G
    zrows = jnp.zeros((L, D), jnp.float32)

    g_flat = gaussian_pool.reshape(-1)

    mesh = plsc.VectorSubcoreMesh(core_axis_name="c", subcore_axis_name="s")
    f32 = jnp.float32
    i32 = jnp.int32
    dma_sems = [pltpu.SemaphoreType.DMA] * 12
    call = pl.kernel(
        _sc_body,
        out_type=[
            jax.ShapeDtypeStruct((M * G,), f32),
            jax.ShapeDtypeStruct((M,), f32),
            jax.ShapeDtypeStruct((M * G,), f32),
            jax.ShapeDtypeStruct((M * G,), f32),
            jax.ShapeDtypeStruct((M, D), f32),
        ],
        mesh=mesh,
        compiler_params=pltpu.CompilerParams(needs_layout_passes=False),
        scratch_types=[
            pltpu.VMEM((22 * L,), f32),
            pltpu.VMEM((CH * G,), i32),
            pltpu.VMEM((CH,), i32),
            pltpu.VMEM((CH,), f32),
            pltpu.VMEM((CH,), f32),
            pltpu.VMEM((L, D), f32),
            pltpu.VMEM((CH * G,), f32), pltpu.VMEM((CH * G,), f32),
            pltpu.VMEM((CH * G,), f32), pltpu.VMEM((CH * G,), f32),
            pltpu.VMEM((CH * G,), f32), pltpu.VMEM((CH * G,), f32),
            pltpu.VMEM((CH * G,), f32), pltpu.VMEM((CH * G,), f32),
            pltpu.VMEM((CH,), f32), pltpu.VMEM((CH,), f32),
        ] + dma_sems,
    )
    pool_u, tag, reu, unc, inst = call(g_flat, instance_feature_pool,
                                       params_b, rowtab, rowm26, zrows)

    anchor_tiled = jnp.tile(anchor[None], (batch_size, 1, 1))
    return (pool_u.reshape(M, G), tag, reu.reshape(M, G), unc.reshape(M, G),
            inst, anchor_tiled)


# speedup vs baseline: 6.4475x; 6.4475x over previous
import jax
import jax.numpy as jnp
from jax import lax
from jax.experimental import pallas as pl
from jax.experimental.pallas import tpu as pltpu
from jax.experimental.pallas import tpu_sc as plsc

L = 16
NW = 32
G = 26
D = 256
CH = 160
NG = CH // L
GG = CH * G // L
M_TOTAL = 100000
NCHUNK = M_TOTAL // CH


def _sc_body(g_hbm, f_hbm, p_hbm, rt_hbm, rm_hbm,
             pool_out, tag_out, reu_out, unc_out, inst_out,
             pv, rtv, rmv, fm_v, um_v,
             g0, g1, go0, go1, r0, r1, u0, u1, f0, f1, t0, t1,
             *sems):
    G_V = [g0, g1]
    GO_V = [go0, go1]
    R_V = [r0, r1]
    U_V = [u0, u1]
    F_V = [f0, f1]
    T_V = [t0, t1]
    SEM = [sems[0:7], sems[7:14]]

    wid = lax.axis_index("s") * 2 + lax.axis_index("c")

    pltpu.sync_copy(p_hbm, pv)
    pltpu.sync_copy(rt_hbm, rtv)
    pltpu.sync_copy(rm_hbm, rmv)

    P = [pv[pl.ds(i * L, L)] for i in range(22)]
    (w00, w01, w02, w03, w10, w11, w12, w13, w20, w21, w22, w23,
     fx, fy, cx0, cy0, n0, n1, n2, f0b, f1b, f2b) = P

    n_my = (NCHUNK - wid + NW - 1) // NW

    def base_of(k):
        return (wid + k * NW) * CH

    def gin_desc(k, b):
        base = base_of(k)
        return pltpu.make_async_copy(g_hbm.at[pl.ds(base * G, CH * G)],
                                     G_V[b], SEM[b][0])

    def fin_desc(k, b):
        base = base_of(k)
        return pltpu.make_async_copy(f_hbm.at[pl.ds(base * D, CH * D)],
                                     F_V[b], SEM[b][1])

    def fout_desc(k, b):
        base = base_of(k)
        return pltpu.make_async_copy(F_V[b], inst_out.at[pl.ds(base * D, CH * D)],
                                     SEM[b][2])

    def out_descs(k, b):
        base = base_of(k)
        return [
            pltpu.make_async_copy(GO_V[b], pool_out.at[pl.ds(base * G, CH * G)],
                                  SEM[b][3]),
            pltpu.make_async_copy(R_V[b], reu_out.at[pl.ds(base * G, CH * G)],
                                  SEM[b][4]),
            pltpu.make_async_copy(U_V[b], unc_out.at[pl.ds(base * G, CH * G)],
                                  SEM[b][5]),
            pltpu.make_async_copy(T_V[b], tag_out.at[pl.ds(base, CH)],
                                  SEM[b][6]),
        ]

    def bf16_round(v):
        u = plsc.bitcast(v, jnp.int32)
        r = u + (jnp.int32(0x7FFF) + ((u >> 16) & 1))
        return plsc.bitcast(r & jnp.int32(-65536), jnp.float32)

    def compute(b):
        @plsc.parallel_loop(0, NG, unroll=2)
        def mask_body(j):
            m26 = rmv[pl.ds(j * L, L)]
            x = plsc.load_gather(G_V[b], [m26])
            y = plsc.load_gather(G_V[b], [m26 + 1])
            z = plsc.load_gather(G_V[b], [m26 + 2])
            xb = bf16_round(x)
            yb = bf16_round(y)
            zb = bf16_round(z)
            cx = xb * w00 + yb * w01 + zb * w02 + w03
            cy = xb * w10 + yb * w11 + zb * w12 + w13
            cz_ = xb * w20 + yb * w21 + zb * w22 + w23
            mask1 = cz_ > 1e-6
            cz = jnp.maximum(cz_, 1e-6)
            pxf = fx * (cx / cz) + cx0
            pyf = fy * (cy / cz) + cy0
            mask2 = ((pxf >= 0.0) & (pxf < 640.0)
                     & (pyf >= 0.0) & (pyf < 480.0))
            in_vox = ((x > n0) & (x < f0b) & (y > n1) & (y < f1b)
                      & (z > n2) & (z < f2b))
            mask_det = mask1 & mask2 & in_vox
            g23 = plsc.load_gather(G_V[b], [m26 + 23])
            g24 = plsc.load_gather(G_V[b], [m26 + 24])
            one = jnp.full((L,), 1.0, jnp.float32)
            half = jnp.full((L,), 0.5, jnp.float32)
            zero = jnp.zeros((L,), jnp.float32)
            plsc.store_scatter(G_V[b], [m26 + 24],
                               jnp.where(in_vox, one, g24))
            T_V[b][pl.ds(j * L, L)] = jnp.where(
                mask_det, jnp.where(g23 == 1.0, half, zero), one)
            fm_v[pl.ds(j * L, L)] = jnp.where(in_vox, one, zero)
            um_v[pl.ds(j * L, L)] = jnp.where(mask_det, zero, one)

        @plsc.parallel_loop(0, GG, unroll=4)
        def g_body(kk):
            off = kk * L
            row = rtv[pl.ds(off, L)]
            fmv = plsc.load_gather(fm_v, [row])
            umv = plsc.load_gather(um_v, [row])
            gv = G_V[b][pl.ds(off, L)]
            GO_V[b][pl.ds(off, L)] = gv
            R_V[b][pl.ds(off, L)] = gv * fmv
            U_V[b][pl.ds(off, L)] = gv * umv

        zero = jnp.zeros((L,), jnp.float32)

        def z_body(j, _):
            fmg = fm_v[pl.ds(j * L, L)]
            zm = fmg == 0.0

            @pl.when(jnp.any(zm))
            def _():
                for lane in range(L):
                    @pl.when(fmg[lane] == 0.0)
                    def _(lane=lane):
                        roff = (j * L + lane) * D
                        for cg in range(D // L):
                            F_V[b][pl.ds(roff + cg * L, L)] = zero

            return 0

        lax.fori_loop(0, NG, z_body, 0)

    gin_desc(0, 0).start()
    fin_desc(0, 0).start()

    def body(i, _):
        for bslot in (0, 1):
            k = i * 2 + bslot

            @pl.when(k < n_my)
            def _(k=k, bslot=bslot):
                gin_desc(k, bslot).wait()
                fin_desc(k, bslot).wait()

                @pl.when(k >= 2)
                def _():
                    for dsc in out_descs(k - 2, bslot):
                        dsc.wait()

                @pl.when(k + 1 < n_my)
                def _():
                    gin_desc(k + 1, 1 - bslot).start()

                compute(bslot)

                @pl.when(k >= 1)
                def _():
                    fout_desc(k - 1, 1 - bslot).wait()

                @pl.when(k + 1 < n_my)
                def _():
                    fin_desc(k + 1, 1 - bslot).start()

                fout_desc(k, bslot).start()
                for dsc in out_descs(k, bslot):
                    dsc.start()

        return 0

    lax.fori_loop(0, (n_my + 1) // 2, body, 0)

    for bslot in (0, 1):
        for k_last, with_fout in ((n_my - 2, False), (n_my - 1, True)):
            @pl.when((k_last >= 0) & (k_last % 2 == bslot))
            def _(k_last=k_last, bslot=bslot, with_fout=with_fout):
                if with_fout:
                    fout_desc(k_last, bslot).wait()
                for dsc in out_descs(k_last, bslot):
                    dsc.wait()


def kernel(gaussian_pool, instance_feature_pool, world2cam, cam_k,
           vox_origin, scene_size, mlvl_img_feat, anchor):
    M = gaussian_pool.shape[0]
    batch_size = mlvl_img_feat.shape[0]
    eps = jnp.float32(0.001)
    near = vox_origin + eps
    far = vox_origin + scene_size - eps
    w_rounded = world2cam[:3, :].astype(jnp.bfloat16).astype(jnp.float32)
    params = jnp.concatenate([
        w_rounded.reshape(-1),
        jnp.stack([cam_k[0, 0], cam_k[1, 1], cam_k[0, 2], cam_k[1, 2]]),
        near, far,
    ]).astype(jnp.float32)
    params_b = jnp.repeat(params, L)

    rowtab = jnp.repeat(jnp.arange(CH, dtype=jnp.int32), G)
    rowm26 = jnp.arange(CH, dtype=jnp.int32) * G

    g_flat = gaussian_pool.reshape(-1)
    f_flat = instance_feature_pool.reshape(-1)

    mesh = plsc.VectorSubcoreMesh(core_axis_name="c", subcore_axis_name="s")
    f32 = jnp.float32
    i32 = jnp.int32
    dma_sems = [pltpu.SemaphoreType.DMA] * 14
    call = pl.kernel(
        _sc_body,
        out_type=[
            jax.ShapeDtypeStruct((M * G,), f32),
            jax.ShapeDtypeStruct((M,), f32),
            jax.ShapeDtypeStruct((M * G,), f32),
            jax.ShapeDtypeStruct((M * G,), f32),
            jax.ShapeDtypeStruct((M * D,), f32),
        ],
        mesh=mesh,
        compiler_params=pltpu.CompilerParams(needs_layout_passes=False),
        scratch_types=[
            pltpu.VMEM((22 * L,), f32),
            pltpu.VMEM((CH * G,), i32),
            pltpu.VMEM((CH,), i32),
            pltpu.VMEM((CH,), f32),
            pltpu.VMEM((CH,), f32),
            pltpu.VMEM((CH * G,), f32), pltpu.VMEM((CH * G,), f32),
            pltpu.VMEM((CH * G,), f32), pltpu.VMEM((CH * G,), f32),
            pltpu.VMEM((CH * G,), f32), pltpu.VMEM((CH * G,), f32),
            pltpu.VMEM((CH * G,), f32), pltpu.VMEM((CH * G,), f32),
            pltpu.VMEM((CH * D,), f32), pltpu.VMEM((CH * D,), f32),
            pltpu.VMEM((CH,), f32), pltpu.VMEM((CH,), f32),
        ] + dma_sems,
    )
    pool_u, tag, reu, unc, inst = call(g_flat, f_flat, params_b, rowtab, rowm26)

    anchor_tiled = jnp.tile(anchor[None], (batch_size, 1, 1))
    return (pool_u.reshape(M, G), tag, reu.reshape(M, G), unc.reshape(M, G),
            inst.reshape(M, D), anchor_tiled)


# speedup vs baseline: 11.1488x; 1.7292x over previous
import jax
import jax.numpy as jnp
from jax import lax
from jax.experimental import pallas as pl
from jax.experimental.pallas import tpu as pltpu
from jax.experimental.pallas import tpu_sc as plsc

L = 16
NW = 32
G = 26
D = 256
CH = 80
NG = CH // L
GG = CH * G // L
M_TOTAL = 100000
NCHUNK = M_TOTAL // CH


def _sc_body(g_hbm, f_hbm, p_hbm, rt_hbm, ct_hbm,
             pool_out, tag_out, reu_out, unc_out, inst_out,
             pv, rtv, ctv, fm_v, um_v,
             g0, g1, go0, go1, r0, r1, u0, u1, f0, f1, t0, t1,
             *sems):
    G_V = [g0, g1]
    GO_V = [go0, go1]
    R_V = [r0, r1]
    U_V = [u0, u1]
    F_V = [f0, f1]
    T_V = [t0, t1]
    SEM = [sems[0:7], sems[7:14]]

    wid = lax.axis_index("s") * 2 + lax.axis_index("c")

    pltpu.sync_copy(p_hbm, pv)
    pltpu.sync_copy(rt_hbm, rtv)
    pltpu.sync_copy(ct_hbm, ctv)

    P = [pv[pl.ds(i * L, L)] for i in range(22)]
    (w00, w01, w02, w03, w10, w11, w12, w13, w20, w21, w22, w23,
     fx, fy, cx0, cy0, n0, n1, n2, f0b, f1b, f2b) = P

    n_my = (NCHUNK - wid + NW - 1) // NW
    iota = lax.iota(jnp.int32, L)

    def base_of(k):
        return (wid + k * NW) * CH

    def gin_desc(k, b):
        return pltpu.make_async_copy(g_hbm.at[pl.ds(base_of(k), CH)],
                                     G_V[b], SEM[b][0])

    def fin_desc(k, b):
        return pltpu.make_async_copy(f_hbm.at[pl.ds(base_of(k), CH)],
                                     F_V[b], SEM[b][1])

    def fout_desc(k, b):
        return pltpu.make_async_copy(F_V[b], inst_out.at[pl.ds(base_of(k), CH)],
                                     SEM[b][2])

    def out_descs(k, b):
        base = base_of(k)
        return [
            pltpu.make_async_copy(GO_V[b], pool_out.at[pl.ds(base, CH)],
                                  SEM[b][3]),
            pltpu.make_async_copy(R_V[b], reu_out.at[pl.ds(base, CH)],
                                  SEM[b][4]),
            pltpu.make_async_copy(U_V[b], unc_out.at[pl.ds(base, CH)],
                                  SEM[b][5]),
            pltpu.make_async_copy(T_V[b], tag_out.at[pl.ds(base, CH)],
                                  SEM[b][6]),
        ]

    def bf16_round(v):
        u = plsc.bitcast(v, jnp.int32)
        r = u + (jnp.int32(0x7FFF) + ((u >> 16) & 1))
        return plsc.bitcast(r & jnp.int32(-65536), jnp.float32)

    def col(c):
        return jnp.full((L,), c, jnp.int32)

    def compute(b):
        @plsc.parallel_loop(0, NG, unroll=2)
        def mask_body(j):
            rows = j * L + iota
            x = plsc.load_gather(G_V[b], [rows, col(0)])
            y = plsc.load_gather(G_V[b], [rows, col(1)])
            z = plsc.load_gather(G_V[b], [rows, col(2)])
            xb = bf16_round(x)
            yb = bf16_round(y)
            zb = bf16_round(z)
            cx = xb * w00 + yb * w01 + zb * w02 + w03
            cy = xb * w10 + yb * w11 + zb * w12 + w13
            cz_ = xb * w20 + yb * w21 + zb * w22 + w23
            mask1 = cz_ > 1e-6
            cz = jnp.maximum(cz_, 1e-6)
            pxf = fx * (cx / cz) + cx0
            pyf = fy * (cy / cz) + cy0
            mask2 = ((pxf >= 0.0) & (pxf < 640.0)
                     & (pyf >= 0.0) & (pyf < 480.0))
            in_vox = ((x > n0) & (x < f0b) & (y > n1) & (y < f1b)
                      & (z > n2) & (z < f2b))
            mask_det = mask1 & mask2 & in_vox
            g23 = plsc.load_gather(G_V[b], [rows, col(23)])
            g24 = plsc.load_gather(G_V[b], [rows, col(24)])
            one = jnp.full((L,), 1.0, jnp.float32)
            half = jnp.full((L,), 0.5, jnp.float32)
            zero = jnp.zeros((L,), jnp.float32)
            plsc.store_scatter(G_V[b], [rows, col(24)],
                               jnp.where(in_vox, one, g24))
            T_V[b][pl.ds(j * L, L)] = jnp.where(
                mask_det, jnp.where(g23 == 1.0, half, zero), one)
            fm_v[pl.ds(j * L, L)] = jnp.where(in_vox, one, zero)
            um_v[pl.ds(j * L, L)] = jnp.where(mask_det, zero, one)

        @plsc.parallel_loop(0, GG, unroll=4)
        def g_body(kk):
            off = kk * L
            rowv = rtv[pl.ds(off, L)]
            colv = ctv[pl.ds(off, L)]
            fmv = plsc.load_gather(fm_v, [rowv])
            umv = plsc.load_gather(um_v, [rowv])
            gv = plsc.load_gather(G_V[b], [rowv, colv])
            plsc.store_scatter(GO_V[b], [rowv, colv], gv)
            plsc.store_scatter(R_V[b], [rowv, colv], gv * fmv)
            plsc.store_scatter(U_V[b], [rowv, colv], gv * umv)

        zero = jnp.zeros((L,), jnp.float32)

        def z_body(j, _):
            fmg = fm_v[pl.ds(j * L, L)]
            zm = fmg == 0.0

            @pl.when(jnp.any(zm))
            def _():
                for lane in range(L):
                    @pl.when(fmg[lane] == 0.0)
                    def _(lane=lane):
                        rowb = col(j * L + lane)
                        for cg in range(D // L):
                            plsc.store_scatter(F_V[b], [rowb, cg * L + iota],
                                               zero)

            return 0

        lax.fori_loop(0, NG, z_body, 0)

    gin_desc(0, 0).start()
    fin_desc(0, 0).start()

    def body(i, _):
        for bslot in (0, 1):
            k = i * 2 + bslot

            @pl.when(k < n_my)
            def _(k=k, bslot=bslot):
                gin_desc(k, bslot).wait()
                fin_desc(k, bslot).wait()

                @pl.when(k >= 2)
                def _():
                    for dsc in out_descs(k - 2, bslot):
                        dsc.wait()

                @pl.when(k + 1 < n_my)
                def _():
                    gin_desc(k + 1, 1 - bslot).start()

                compute(bslot)

                @pl.when(k >= 1)
                def _():
                    fout_desc(k - 1, 1 - bslot).wait()

                @pl.when(k + 1 < n_my)
                def _():
                    fin_desc(k + 1, 1 - bslot).start()

                fout_desc(k, bslot).start()
                for dsc in out_descs(k, bslot):
                    dsc.start()

        return 0

    lax.fori_loop(0, (n_my + 1) // 2, body, 0)

    for bslot in (0, 1):
        for k_last, with_fout in ((n_my - 2, False), (n_my - 1, True)):
            @pl.when((k_last >= 0) & (k_last % 2 == bslot))
            def _(k_last=k_last, bslot=bslot, with_fout=with_fout):
                if with_fout:
                    fout_desc(k_last, bslot).wait()
                for dsc in out_descs(k_last, bslot):
                    dsc.wait()


def kernel(gaussian_pool, instance_feature_pool, world2cam, cam_k,
           vox_origin, scene_size, mlvl_img_feat, anchor):
    M = gaussian_pool.shape[0]
    batch_size = mlvl_img_feat.shape[0]
    eps = jnp.float32(0.001)
    near = vox_origin + eps
    far = vox_origin + scene_size - eps
    w_rounded = world2cam[:3, :].astype(jnp.bfloat16).astype(jnp.float32)
    params = jnp.concatenate([
        w_rounded.reshape(-1),
        jnp.stack([cam_k[0, 0], cam_k[1, 1], cam_k[0, 2], cam_k[1, 2]]),
        near, far,
    ]).astype(jnp.float32)
    params_b = jnp.repeat(params, L)

    rowtab = jnp.repeat(jnp.arange(CH, dtype=jnp.int32), G)
    coltab = jnp.tile(jnp.arange(G, dtype=jnp.int32), CH)

    mesh = plsc.VectorSubcoreMesh(core_axis_name="c", subcore_axis_name="s")
    f32 = jnp.float32
    i32 = jnp.int32
    dma_sems = [pltpu.SemaphoreType.DMA] * 14
    call = pl.kernel(
        _sc_body,
        out_type=[
            jax.ShapeDtypeStruct((M, G), f32),
            jax.ShapeDtypeStruct((M,), f32),
            jax.ShapeDtypeStruct((M, G), f32),
            jax.ShapeDtypeStruct((M, G), f32),
            jax.ShapeDtypeStruct((M, D), f32),
        ],
        mesh=mesh,
        compiler_params=pltpu.CompilerParams(needs_layout_passes=False),
        scratch_types=[
            pltpu.VMEM((22 * L,), f32),
            pltpu.VMEM((CH * G,), i32),
            pltpu.VMEM((CH * G,), i32),
            pltpu.VMEM((CH,), f32),
            pltpu.VMEM((CH,), f32),
            pltpu.VMEM((CH, G), f32), pltpu.VMEM((CH, G), f32),
            pltpu.VMEM((CH, G), f32), pltpu.VMEM((CH, G), f32),
            pltpu.VMEM((CH, G), f32), pltpu.VMEM((CH, G), f32),
            pltpu.VMEM((CH, G), f32), pltpu.VMEM((CH, G), f32),
            pltpu.VMEM((CH, D), f32), pltpu.VMEM((CH, D), f32),
            pltpu.VMEM((CH,), f32), pltpu.VMEM((CH,), f32),
        ] + dma_sems,
    )
    pool_u, tag, reu, unc, inst = call(gaussian_pool, instance_feature_pool,
                                       params_b, rowtab, coltab)

    anchor_tiled = jnp.tile(anchor[None], (batch_size, 1, 1))
    return (pool_u, tag, reu, unc, inst, anchor_tiled)
